# baseline (speedup 1.0000x reference)
import functools

import jax
import jax.numpy as jnp
from jax import lax
from jax.experimental import pallas as pl
from jax.experimental.pallas import tpu as pltpu
from jax.experimental.pallas import tpu_sc as plsc

D = 1024
S = 512
N_TOK = 32768
NC = 2
NW = 32
NSPLIT = 4
TOK_SPLIT = N_TOK // NSPLIT
TOK_PER_W = TOK_SPLIT // NW
C = 16
NCH = TOK_PER_W // C
NBUF = 3
R = 512


def _sc_gather_body(ids, tok_tab, out, ids_v, idx3, tok3, semg, sems):
    wid = lax.axis_index("s") * NC + lax.axis_index("c")
    w_base = wid * TOK_PER_W
    pltpu.sync_copy(ids.at[pl.ds(w_base, TOK_PER_W)], ids_v)

    def issue_g(cc, b):
        idx3[b, ...] = ids_v[pl.ds(cc * C, C)]
        pltpu.async_copy(tok_tab.at[idx3.at[b]], tok3.at[b], semg.at[b])

    def wait_g(b):
        pltpu.make_async_copy(tok_tab.at[idx3.at[b]], tok3.at[b],
                              semg.at[b]).wait()

    def wait_s(b):
        pltpu.make_async_copy(tok3.at[b], out.at[pl.ds(w_base, C)],
                              sems.at[b]).wait()

    issue_g(0, 0)
    issue_g(1, 1)

    def chunk_body(cc, carry):
        b, b1, b2 = carry

        @pl.when(cc + 2 < NCH)
        def _():
            issue_g(cc + 2, b2)

        wait_g(b)
        pltpu.sync_copy(tok3.at[b], out.at[pl.ds(w_base + cc * C, C)])
        return (b1, b2, b)

    lax.fori_loop(0, NCH, chunk_body,
                  (jnp.int32(0), jnp.int32(1), jnp.int32(2)))


_sc_gather = functools.partial(
    pl.kernel,
    mesh=plsc.VectorSubcoreMesh(core_axis_name="c", subcore_axis_name="s"),
    out_type=jax.ShapeDtypeStruct((TOK_SPLIT, D), jnp.float32),
    scratch_types=(
        [pltpu.VMEM((TOK_PER_W,), jnp.int32)]
        + [pltpu.VMEM((NBUF, C), jnp.int32)]
        + [pltpu.VMEM((NBUF, C, D), jnp.float32)]
        + [pltpu.SemaphoreType.DMA((NBUF,))] * 2
    ),
)(_sc_gather_body)


def _ln_body(emb_ref, pos_ref, seg_ref, ttf_ref, gam_ref, bet_ref, out_ref):
    s0 = seg_ref[0]
    sd = seg_ref[1] - seg_ref[0]
    x = emb_ref[...] + pos_ref[...] + s0 + ttf_ref[...] * sd
    mean = jnp.mean(x, axis=1, keepdims=True)
    ex2 = jnp.mean(x * x, axis=1, keepdims=True)
    var = ex2 - mean * mean
    rstd = lax.rsqrt(var + jnp.float32(1e-12))
    out_ref[...] = (x - mean) * rstd * gam_ref[...] + bet_ref[...]


def _tc_ln(emb, pos_table, seg_table, ttf, ln_gamma, ln_beta):
    grid = (TOK_SPLIT // R,)
    pos_blocks = S // R
    return pl.pallas_call(
        _ln_body,
        grid=grid,
        in_specs=[
            pl.BlockSpec((R, D), lambda i: (i, 0)),
            pl.BlockSpec((R, D), lambda i: (i % pos_blocks, 0)),
            pl.BlockSpec((2, 1, D), lambda i: (0, 0, 0)),
            pl.BlockSpec((R, 1), lambda i: (i, 0)),
            pl.BlockSpec((1, D), lambda i: (0, 0)),
            pl.BlockSpec((1, D), lambda i: (0, 0)),
        ],
        out_specs=pl.BlockSpec((R, D), lambda i: (i, 0)),
        out_shape=jax.ShapeDtypeStruct((TOK_SPLIT, D), jnp.float32),
    )(emb, pos_table, seg_table.reshape(2, 1, D), ttf,
      ln_gamma.reshape(1, D), ln_beta.reshape(1, D))


def kernel(input_ids, token_type_ids, tok_table, pos_table, seg_table,
           ln_gamma, ln_beta):
    B, Sq = input_ids.shape
    ids = input_ids.reshape(NSPLIT, TOK_SPLIT)
    ttf = token_type_ids.reshape(NSPLIT, TOK_SPLIT, 1).astype(jnp.float32)
    outs = []
    for h in range(NSPLIT):
        emb = _sc_gather(ids[h], tok_table)
        outs.append(_tc_ln(emb, pos_table, seg_table, ttf[h],
                           ln_gamma, ln_beta))
    return jnp.concatenate(outs, axis=0).reshape(B, Sq, D)


# speedup vs baseline: 1.1330x; 1.1330x over previous
import functools

import jax
import jax.numpy as jnp
from jax import lax
from jax.experimental import pallas as pl
from jax.experimental.pallas import tpu as pltpu
from jax.experimental.pallas import tpu_sc as plsc

D = 1024
S = 512
N_TOK = 32768
NC = 2
NW = 32
NSPLIT = 2
TOK_SPLIT = N_TOK // NSPLIT
TOK_PER_W = TOK_SPLIT // NW
C = 16
NCH = TOK_PER_W // C
NBUF = 3
R = 1024


def _sc_gather_body(ids, tok_tab, out, ids_v, idx3, tok3, semg, sems):
    wid = lax.axis_index("s") * NC + lax.axis_index("c")
    w_base = wid * TOK_PER_W
    pltpu.sync_copy(ids.at[pl.ds(w_base, TOK_PER_W)], ids_v)

    def issue_g(cc, b):
        idx3[b, ...] = ids_v[pl.ds(cc * C, C)]
        pltpu.async_copy(tok_tab.at[idx3.at[b]], tok3.at[b], semg.at[b])

    def wait_g(b):
        pltpu.make_async_copy(tok_tab.at[idx3.at[b]], tok3.at[b],
                              semg.at[b]).wait()

    def wait_s(b):
        pltpu.make_async_copy(tok3.at[b], out.at[pl.ds(w_base, C)],
                              sems.at[b]).wait()

    issue_g(0, 0)
    issue_g(1, 1)

    def chunk_body(cc, carry):
        b, b1, b2 = carry

        @pl.when(cc + 2 < NCH)
        def _():
            issue_g(cc + 2, b2)

        wait_g(b)
        pltpu.sync_copy(tok3.at[b], out.at[pl.ds(w_base + cc * C, C)])
        return (b1, b2, b)

    lax.fori_loop(0, NCH, chunk_body,
                  (jnp.int32(0), jnp.int32(1), jnp.int32(2)))


_sc_gather = functools.partial(
    pl.kernel,
    mesh=plsc.VectorSubcoreMesh(core_axis_name="c", subcore_axis_name="s"),
    out_type=jax.ShapeDtypeStruct((TOK_SPLIT, D), jnp.float32),
    scratch_types=(
        [pltpu.VMEM((TOK_PER_W,), jnp.int32)]
        + [pltpu.VMEM((NBUF, C), jnp.int32)]
        + [pltpu.VMEM((NBUF, C, D), jnp.float32)]
        + [pltpu.SemaphoreType.DMA((NBUF,))] * 2
    ),
)(_sc_gather_body)


def _ln_body(emb_ref, pos_ref, seg_ref, ttf_ref, gam_ref, bet_ref, out_ref):
    s0 = seg_ref[0]
    sd = seg_ref[1] - seg_ref[0]
    x = emb_ref[...] + pos_ref[...] + s0 + ttf_ref[...] * sd
    mean = jnp.mean(x, axis=1, keepdims=True)
    ex2 = jnp.mean(x * x, axis=1, keepdims=True)
    var = ex2 - mean * mean
    rstd = lax.rsqrt(var + jnp.float32(1e-12))
    out_ref[...] = (x - mean) * rstd * gam_ref[...] + bet_ref[...]


def _tc_ln(emb, pos_table, seg_table, ttf, ln_gamma, ln_beta):
    grid = (TOK_SPLIT // R,)
    pos_blocks = S // R
    return pl.pallas_call(
        _ln_body,
        grid=grid,
        in_specs=[
            pl.BlockSpec((R, D), lambda i: (i, 0)),
            pl.BlockSpec((R, D), lambda i: (i % pos_blocks, 0)),
            pl.BlockSpec((2, 1, D), lambda i: (0, 0, 0)),
            pl.BlockSpec((R, 1), lambda i: (i, 0)),
            pl.BlockSpec((1, D), lambda i: (0, 0)),
            pl.BlockSpec((1, D), lambda i: (0, 0)),
        ],
        out_specs=pl.BlockSpec((R, D), lambda i: (i, 0)),
        out_shape=jax.ShapeDtypeStruct((TOK_SPLIT, D), jnp.float32),
    )(emb, pos_table, seg_table.reshape(2, 1, D), ttf,
      ln_gamma.reshape(1, D), ln_beta.reshape(1, D))


def kernel(input_ids, token_type_ids, tok_table, pos_table, seg_table,
           ln_gamma, ln_beta):
    B, Sq = input_ids.shape
    ids = input_ids.reshape(NSPLIT, TOK_SPLIT)
    ttf = token_type_ids.reshape(NSPLIT, TOK_SPLIT, 1).astype(jnp.float32)
    outs = []
    for h in range(NSPLIT):
        emb = _sc_gather(ids[h], tok_table)
        outs.append(_tc_ln(emb, pos_table, seg_table, ttf[h],
                           ln_gamma, ln_beta))
    return jnp.concatenate(outs, axis=0).reshape(B, Sq, D)
